# baseline (speedup 1.0000x reference)
import functools

import jax
import jax.numpy as jnp
from jax import lax
from jax.experimental import pallas as pl
from jax.experimental.pallas import tpu as pltpu
from jax.experimental.pallas import tpu_sc as plsc

_B, _S, _C = 1024, 50, 32
_K = 128
_SB = 640


def _spmm_body(nnz_per_tile, trip_view, vals_hbm, w_hbm, out_hbm,
               accum, tbuf, cvals, rowv, gbuf, sem0, sem1, semt0, semt1):
    cid = lax.axis_index("c")
    sid = lax.axis_index("s")
    wid = cid * 16 + sid
    base = wid * nnz_per_tile

    rows_per_tile = _B * _S // 16
    zv = jnp.zeros((16,), jnp.float32)
    def zero_gbuf(i, _):
        gbuf[0, i, pl.ds(0, 16)] = zv
        gbuf[0, i, pl.ds(16, 16)] = zv
        return 0
    lax.fori_loop(0, _K, zero_gbuf, 0)
    def zero_accum(i, _):
        pltpu.sync_copy(gbuf.at[0],
                        accum.at[pl.ds(sid * rows_per_tile + i * _K, _K)])
        return 0
    lax.fori_loop(0, rows_per_tile // _K, zero_accum, 0)
    plsc.subcore_barrier()

    n_super = nnz_per_tile // _SB
    cpb = _SB // _K

    def start_load(g, tslot):
        off = base + g * _SB
        sem = semt0 if tslot == 0 else semt1
        pltpu.async_copy(trip_view.at[:, pl.ds(off, _SB)], tbuf.at[tslot], sem)
        pltpu.async_copy(vals_hbm.at[pl.ds(off, _SB)], cvals.at[tslot], sem)

    def wait_load(tslot):
        sem = semt0 if tslot == 0 else semt1
        pltpu.make_async_copy(trip_view.at[:, pl.ds(0, _SB)], tbuf.at[tslot],
                              sem).wait()
        pltpu.make_async_copy(vals_hbm.at[pl.ds(0, _SB)], cvals.at[tslot],
                              sem).wait()

    def start_gather(tslot, k, slot):
        pltpu.async_copy(
            w_hbm.at[tbuf.at[tslot, 1, pl.ds(k * _K, _K)]], gbuf.at[slot],
            sem0 if slot == 0 else sem1)

    def wait_gather(slot):
        pltpu.make_async_copy(w_hbm.at[tbuf.at[0, 1, pl.ds(0, _K)]],
                              gbuf.at[slot],
                              sem0 if slot == 0 else sem1).wait()

    def process_chunk(tslot, k, slot):
        for v in range(_K // 16):
            rv = tbuf[tslot, 0, pl.ds(k * _K + v * 16, 16)]
            q = ((rv.astype(jnp.float32) + 0.5) * (1.0 / _S)).astype(jnp.int32)
            m = rv - q * _S
            rowv[pl.ds(v * 16, 16)] = m * _B + q
        dn = lax.GatherDimensionNumbers(offset_dims=(),
                                        collapsed_slice_dims=(0,),
                                        start_index_map=(0,))
        def scale(jg, _):
            vv = cvals[tslot, pl.ds(k * _K + jg * 16, 16)]
            for u in range(16):
                j = jg * 16 + u
                vj = lax.gather(vv, jnp.full((16, 1), u, jnp.int32), dn, (1,),
                                mode=lax.GatherScatterMode.PROMISE_IN_BOUNDS)
                gbuf[slot, j, pl.ds(0, 16)] = gbuf[slot, j, pl.ds(0, 16)] * vj
                gbuf[slot, j, pl.ds(16, 16)] = gbuf[slot, j, pl.ds(16, 16)] * vj
            return 0
        lax.fori_loop(0, _K // 16, scale, 0)
        pltpu.sync_copy(gbuf.at[slot], accum.at[rowv], add=True)

    def pipeline(tslot):
        start_gather(tslot, 0, 0)
        def pair(p, _):
            start_gather(tslot, 2 * p + 1, 1)
            wait_gather(0)
            process_chunk(tslot, 2 * p, 0)
            start_gather(tslot, 2 * p + 2, 0)
            wait_gather(1)
            process_chunk(tslot, 2 * p + 1, 1)
            return 0
        lax.fori_loop(0, (cpb - 1) // 2, pair, 0)
        if cpb % 2 == 1:
            wait_gather(0)
            process_chunk(tslot, cpb - 1, 0)
        else:
            start_gather(tslot, cpb - 1, 1)
            wait_gather(0)
            process_chunk(tslot, cpb - 2, 0)
            wait_gather(1)
            process_chunk(tslot, cpb - 1, 1)

    start_load(0, 0)
    def super2(g2, _):
        g = 2 * g2
        wait_load(0)
        @pl.when(g + 1 < n_super)
        def _():
            start_load(g + 1, 1)
        pipeline(0)
        @pl.when(g + 1 < n_super)
        def _():
            wait_load(1)
            @pl.when(g + 2 < n_super)
            def _():
                start_load(g + 2, 0)
            pipeline(1)
        return 0
    lax.fori_loop(0, (n_super + 1) // 2, super2, 0)
    plsc.subcore_barrier()

    def writeback(i, _):
        r = sid * rows_per_tile + i * _K
        pltpu.sync_copy(accum.at[pl.ds(r, _K)], gbuf.at[0])
        pltpu.sync_copy(gbuf.at[0], out_hbm.at[cid, pl.ds(r, _K)])
        return 0
    lax.fori_loop(0, rows_per_tile // _K, writeback, 0)


def _make_spmm(nnz):
    nnz_per_tile = nnz // 32
    mesh = plsc.VectorSubcoreMesh(core_axis_name="c", subcore_axis_name="s")
    return pl.kernel(
        functools.partial(_spmm_body, nnz_per_tile),
        out_type=jax.ShapeDtypeStruct((2, _B * _S, _C), jnp.float32),
        mesh=mesh,
        compiler_params=pltpu.CompilerParams(use_tc_tiling_on_sc=False),
        scratch_types=[
            pltpu.VMEM_SHARED((_B * _S, _C), jnp.float32),
            pltpu.VMEM((2, 2, _SB), jnp.int32),
            pltpu.VMEM((2, _SB), jnp.float32),
            pltpu.VMEM((_K,), jnp.int32),
            pltpu.VMEM((2, _K, _C), jnp.float32),
            pltpu.SemaphoreType.DMA,
            pltpu.SemaphoreType.DMA,
            pltpu.SemaphoreType.DMA,
            pltpu.SemaphoreType.DMA,
        ],
    )


_SPB = 10


def _crf_body(p0_ref, p1_ref, b_ref, trans_ref, start_ref, startc_ref,
              endc_ref, t_ref, out_ref, alpha_ref, oh_prev_ref, score_ref,
              expt_ref):
    g = pl.program_id(0)
    ng = pl.num_programs(0)
    eye = (lax.broadcasted_iota(jnp.int32, (_C, _C), 0)
           == lax.broadcasted_iota(jnp.int32, (_C, _C), 1)).astype(jnp.float32)

    def pieces(sl):
        em = p0_ref[sl] + p1_ref[sl] + b_ref[...]
        oht = (lax.broadcasted_iota(jnp.int32, (_C, _B), 0) == t_ref[sl]
               ).astype(jnp.float32)
        gold_em = jnp.sum(
            lax.dot_general(oht, em, (((1,), (0,)), ((), ())),
                            preferred_element_type=jnp.float32) * eye)
        return em, oht, gold_em

    def step(em, oht, gold_em):
        cnt = lax.dot_general(oh_prev_ref[...], oht, (((1,), (1,)), ((), ())),
                              preferred_element_type=jnp.float32)
        score_ref[0] = (score_ref[0] + jnp.sum(cnt * trans_ref[...]) + gold_em)
        oh_prev_ref[...] = oht
        a = alpha_ref[...]
        m = jnp.max(a, axis=1, keepdims=True)
        e = jnp.exp(a - m)
        sv = lax.dot_general(e, expt_ref[...], (((1,), (0,)), ((), ())),
                             preferred_element_type=jnp.float32)
        alpha_ref[...] = jnp.log(sv) + m + em

    for sl in range(_SPB):
        em, oht, gold_em = pieces(sl)
        if sl == 0:
            @pl.when(g == 0)
            def _init():
                expt_ref[...] = jnp.exp(trans_ref[...])
                alpha_ref[...] = start_ref[...] + em
                score_ref[0] = gold_em + jnp.sum(oht * startc_ref[...])
                oh_prev_ref[...] = oht

            @pl.when(g > 0)
            def _step0():
                step(em, oht, gold_em)
        else:
            step(em, oht, gold_em)
        last_oht = oht

    @pl.when(g == ng - 1)
    def _fin():
        score = score_ref[0] + jnp.sum(last_oht * endc_ref[...])
        a = alpha_ref[...] + jnp.sum(endc_ref[...] * eye, axis=0,
                                     keepdims=True)
        m = jnp.max(a, axis=1, keepdims=True)
        denom = jnp.log(jnp.sum(jnp.exp(a - m), axis=1, keepdims=True)) + m
        out_ref[...] = jnp.reshape(jnp.sum(denom) - score, (1, 1))


_crf_call = pl.pallas_call(
    _crf_body,
    grid=(_S // _SPB,),
    in_specs=[
        pl.BlockSpec((_SPB, _B, _C), lambda g: (g, 0, 0)),
        pl.BlockSpec((_SPB, _B, _C), lambda g: (g, 0, 0)),
        pl.BlockSpec((1, _C), lambda g: (0, 0)),
        pl.BlockSpec((_C, _C), lambda g: (0, 0)),
        pl.BlockSpec((1, _C), lambda g: (0, 0)),
        pl.BlockSpec((_C, 1), lambda g: (0, 0)),
        pl.BlockSpec((_C, 1), lambda g: (0, 0)),
        pl.BlockSpec((_SPB, 1, _B), lambda g: (g, 0, 0)),
    ],
    out_specs=pl.BlockSpec((1, 1), lambda g: (0, 0)),
    out_shape=jax.ShapeDtypeStruct((1, 1), jnp.float32),
    scratch_shapes=[
        pltpu.VMEM((_B, _C), jnp.float32),
        pltpu.VMEM((_C, _B), jnp.float32),
        pltpu.SMEM((1,), jnp.float32),
        pltpu.VMEM((_C, _C), jnp.float32),
    ],
    compiler_params=pltpu.CompilerParams(
        dimension_semantics=("arbitrary",)),
)


def kernel(inputs_rows, inputs_cols, inputs_vals, W, b, transitions,
           start_transitions, end_transitions, targets, mask):
    nnz = inputs_rows.shape[0]
    trip = jnp.stack([inputs_rows.astype(jnp.int32),
                      inputs_cols.astype(jnp.int32)])
    parts = _make_spmm(nnz)(trip, inputs_vals, W)
    p = parts.reshape(2, _S, _B, _C)
    loss = _crf_call(
        p[0], p[1], b.reshape(1, _C), transitions,
        start_transitions.reshape(1, _C), start_transitions.reshape(_C, 1),
        end_transitions.reshape(_C, 1),
        targets.astype(jnp.int32).T.reshape(_S, 1, _B))
    return loss[0, 0]


# speedup vs baseline: 1.0747x; 1.0747x over previous
import functools

import jax
import jax.numpy as jnp
from jax import lax
from jax.experimental import pallas as pl
from jax.experimental.pallas import tpu as pltpu
from jax.experimental.pallas import tpu_sc as plsc

_B, _S, _C = 1024, 50, 32
_K = 64
_SB = 256


def _spmm_body(nnz_per_tile, trip_view, vals_hbm, w_hbm, out_hbm,
               accum, tbuf, cvals, rowv, gbuf,
               gsems, scsems, semt0, semt1):
    cid = lax.axis_index("c")
    sid = lax.axis_index("s")
    wid = cid * 16 + sid
    base = wid * nnz_per_tile

    rows_per_tile = _B * _S // 16
    zv = jnp.zeros((16,), jnp.float32)
    def zero_gbuf(i, _):
        gbuf[0, i, pl.ds(0, 16)] = zv
        gbuf[0, i, pl.ds(16, 16)] = zv
        return 0
    lax.fori_loop(0, _K, zero_gbuf, 0)
    def zero_accum(i, _):
        pltpu.sync_copy(gbuf.at[0],
                        accum.at[pl.ds(sid * rows_per_tile + i * _K, _K)])
        return 0
    lax.fori_loop(0, rows_per_tile // _K, zero_accum, 0)
    plsc.subcore_barrier()

    n_super = nnz_per_tile // _SB
    cpb = _SB // _K

    def start_load(g, tslot):
        off = base + g * _SB
        sem = semt0 if tslot == 0 else semt1
        pltpu.async_copy(trip_view.at[:, pl.ds(off, _SB)], tbuf.at[tslot], sem)
        pltpu.async_copy(vals_hbm.at[pl.ds(off, _SB)], cvals.at[tslot], sem)

    def wait_load(tslot):
        sem = semt0 if tslot == 0 else semt1
        pltpu.make_async_copy(trip_view.at[:, pl.ds(0, _SB)], tbuf.at[tslot],
                              sem).wait()
        pltpu.make_async_copy(vals_hbm.at[pl.ds(0, _SB)], cvals.at[tslot],
                              sem).wait()

    def start_gather(tslot, k):
        pltpu.async_copy(
            w_hbm.at[tbuf.at[tslot, 1, pl.ds(k * _K, _K)]], gbuf.at[k],
            gsems[k])

    def wait_gather(k):
        pltpu.make_async_copy(w_hbm.at[tbuf.at[0, 1, pl.ds(0, _K)]],
                              gbuf.at[k], gsems[k]).wait()

    def start_scatter(k):
        pltpu.async_copy(gbuf.at[k], accum.at[rowv.at[k]], scsems[k],
                         add=True)

    def wait_scatter(k):
        pltpu.make_async_copy(gbuf.at[k], accum.at[rowv.at[k]],
                              scsems[k]).wait()

    dn = lax.GatherDimensionNumbers(offset_dims=(),
                                    collapsed_slice_dims=(0,),
                                    start_index_map=(0,))

    def process_chunk(tslot, k):
        for v in range(_K // 16):
            rv = tbuf[tslot, 0, pl.ds(k * _K + v * 16, 16)]
            q = ((rv.astype(jnp.float32) + 0.5) * (1.0 / _S)).astype(jnp.int32)
            m = rv - q * _S
            rowv[k, pl.ds(v * 16, 16)] = m * _B + q
        def scale(jg, _):
            vv = cvals[tslot, pl.ds(k * _K + jg * 16, 16)]
            for u in range(16):
                j = jg * 16 + u
                vj = lax.gather(vv, jnp.full((16, 1), u, jnp.int32), dn, (1,),
                                mode=lax.GatherScatterMode.PROMISE_IN_BOUNDS)
                gbuf[k, j, pl.ds(0, 16)] = gbuf[k, j, pl.ds(0, 16)] * vj
                gbuf[k, j, pl.ds(16, 16)] = gbuf[k, j, pl.ds(16, 16)] * vj
            return 0
        lax.fori_loop(0, _K // 16, scale, 0)

    def pipeline(tslot, guard):
        for k in range(cpb):
            if guard is None:
                wait_scatter(k)
            else:
                pl.when(guard)(functools.partial(wait_scatter, k))
            start_gather(tslot, k)
        for k in range(cpb):
            wait_gather(k)
            process_chunk(tslot, k)
            start_scatter(k)

    start_load(0, 0)
    def super2(g2, _):
        g = 2 * g2
        wait_load(0)
        @pl.when(g + 1 < n_super)
        def _():
            start_load(g + 1, 1)
        pipeline(0, g2 > 0)
        @pl.when(g + 1 < n_super)
        def _():
            wait_load(1)
            @pl.when(g + 2 < n_super)
            def _():
                start_load(g + 2, 0)
            pipeline(1, None)
        return 0
    lax.fori_loop(0, (n_super + 1) // 2, super2, 0)
    for k in range(cpb):
        wait_scatter(k)
    plsc.subcore_barrier()

    def writeback(i, _):
        r = sid * rows_per_tile + i * _K
        pltpu.sync_copy(accum.at[pl.ds(r, _K)], gbuf.at[0])
        pltpu.sync_copy(gbuf.at[0], out_hbm.at[cid, pl.ds(r, _K)])
        return 0
    lax.fori_loop(0, rows_per_tile // _K, writeback, 0)


def _make_spmm(nnz):
    nnz_per_tile = nnz // 32
    mesh = plsc.VectorSubcoreMesh(core_axis_name="c", subcore_axis_name="s")
    nslots = _SB // _K
    def body(trip_view, vals_hbm, w_hbm, out_hbm, accum, tbuf, cvals, rowv,
             gbuf, g0, g1, g2, g3, s0, s1, s2, s3, semt0, semt1):
        return _spmm_body(nnz_per_tile, trip_view, vals_hbm, w_hbm, out_hbm,
                          accum, tbuf, cvals, rowv, gbuf,
                          [g0, g1, g2, g3], [s0, s1, s2, s3], semt0, semt1)
    return pl.kernel(
        body,
        out_type=jax.ShapeDtypeStruct((2, _B * _S, _C), jnp.float32),
        mesh=mesh,
        compiler_params=pltpu.CompilerParams(use_tc_tiling_on_sc=False),
        scratch_types=[
            pltpu.VMEM_SHARED((_B * _S, _C), jnp.float32),
            pltpu.VMEM((2, 2, _SB), jnp.int32),
            pltpu.VMEM((2, _SB), jnp.float32),
            pltpu.VMEM((4, _K), jnp.int32),
            pltpu.VMEM((4, _K, _C), jnp.float32),
        ] + [pltpu.SemaphoreType.DMA] * 10,
    )


_SPB = 10


def _crf_body(p0_ref, p1_ref, b_ref, trans_ref, start_ref, startc_ref,
              endc_ref, t_ref, out_ref, alpha_ref, oh_prev_ref, score_ref,
              expt_ref):
    g = pl.program_id(0)
    ng = pl.num_programs(0)
    eye = (lax.broadcasted_iota(jnp.int32, (_C, _C), 0)
           == lax.broadcasted_iota(jnp.int32, (_C, _C), 1)).astype(jnp.float32)

    def pieces(sl):
        em = p0_ref[sl] + p1_ref[sl] + b_ref[...]
        oht = (lax.broadcasted_iota(jnp.int32, (_C, _B), 0) == t_ref[sl]
               ).astype(jnp.float32)
        gold_em = jnp.sum(
            lax.dot_general(oht, em, (((1,), (0,)), ((), ())),
                            preferred_element_type=jnp.float32) * eye)
        return em, oht, gold_em

    def step(em, oht, gold_em):
        cnt = lax.dot_general(oh_prev_ref[...], oht, (((1,), (1,)), ((), ())),
                              preferred_element_type=jnp.float32)
        score_ref[0] = (score_ref[0] + jnp.sum(cnt * trans_ref[...]) + gold_em)
        oh_prev_ref[...] = oht
        a = alpha_ref[...]
        m = jnp.max(a, axis=1, keepdims=True)
        e = jnp.exp(a - m)
        sv = lax.dot_general(e, expt_ref[...], (((1,), (0,)), ((), ())),
                             preferred_element_type=jnp.float32)
        alpha_ref[...] = jnp.log(sv) + m + em

    for sl in range(_SPB):
        em, oht, gold_em = pieces(sl)
        if sl == 0:
            @pl.when(g == 0)
            def _init():
                expt_ref[...] = jnp.exp(trans_ref[...])
                alpha_ref[...] = start_ref[...] + em
                score_ref[0] = gold_em + jnp.sum(oht * startc_ref[...])
                oh_prev_ref[...] = oht

            @pl.when(g > 0)
            def _step0():
                step(em, oht, gold_em)
        else:
            step(em, oht, gold_em)
        last_oht = oht

    @pl.when(g == ng - 1)
    def _fin():
        score = score_ref[0] + jnp.sum(last_oht * endc_ref[...])
        a = alpha_ref[...] + jnp.sum(endc_ref[...] * eye, axis=0,
                                     keepdims=True)
        m = jnp.max(a, axis=1, keepdims=True)
        denom = jnp.log(jnp.sum(jnp.exp(a - m), axis=1, keepdims=True)) + m
        out_ref[...] = jnp.reshape(jnp.sum(denom) - score, (1, 1))


_crf_call = pl.pallas_call(
    _crf_body,
    grid=(_S // _SPB,),
    in_specs=[
        pl.BlockSpec((_SPB, _B, _C), lambda g: (g, 0, 0)),
        pl.BlockSpec((_SPB, _B, _C), lambda g: (g, 0, 0)),
        pl.BlockSpec((1, _C), lambda g: (0, 0)),
        pl.BlockSpec((_C, _C), lambda g: (0, 0)),
        pl.BlockSpec((1, _C), lambda g: (0, 0)),
        pl.BlockSpec((_C, 1), lambda g: (0, 0)),
        pl.BlockSpec((_C, 1), lambda g: (0, 0)),
        pl.BlockSpec((_SPB, 1, _B), lambda g: (g, 0, 0)),
    ],
    out_specs=pl.BlockSpec((1, 1), lambda g: (0, 0)),
    out_shape=jax.ShapeDtypeStruct((1, 1), jnp.float32),
    scratch_shapes=[
        pltpu.VMEM((_B, _C), jnp.float32),
        pltpu.VMEM((_C, _B), jnp.float32),
        pltpu.SMEM((1,), jnp.float32),
        pltpu.VMEM((_C, _C), jnp.float32),
    ],
    compiler_params=pltpu.CompilerParams(
        dimension_semantics=("arbitrary",)),
)


def kernel(inputs_rows, inputs_cols, inputs_vals, W, b, transitions,
           start_transitions, end_transitions, targets, mask):
    nnz = inputs_rows.shape[0]
    trip = jnp.stack([inputs_rows.astype(jnp.int32),
                      inputs_cols.astype(jnp.int32)])
    parts = _make_spmm(nnz)(trip, inputs_vals, W)
    p = parts.reshape(2, _S, _B, _C)
    loss = _crf_call(
        p[0], p[1], b.reshape(1, _C), transitions,
        start_transitions.reshape(1, _C), start_transitions.reshape(_C, 1),
        end_transitions.reshape(_C, 1),
        targets.astype(jnp.int32).T.reshape(_S, 1, _B))
    return loss[0, 0]
